# initial kernel scaffold (unmeasured)
import functools

import jax
import jax.numpy as jnp
from jax import lax
from jax.experimental import pallas as pl
from jax.experimental.pallas import tpu as pltpu

N_DEV = 4
S_LOC = 1024
S_GLB = N_DEV * S_LOC
D = 1024
HQ = 8
DH = 128
SCALE = 0.08838834764831843


def kernel(x, Wq, Wk, Wv, Wo):
    def body(x_ref, wq_ref, wk_ref, wv_ref, wo_ref, out_ref,
             kall, vall, comm, send_sems, recv_sems):
        my = lax.axis_index("i")
        left = lax.rem(my - 1 + N_DEV, N_DEV)
        right = lax.rem(my + 1, N_DEV)

        barrier_sem = pltpu.get_barrier_semaphore()
        for nbr in [left, right]:
            pl.semaphore_signal(
                barrier_sem, inc=1,
                device_id=(nbr,), device_id_type=pl.DeviceIdType.MESH,
            )
        pl.semaphore_wait(barrier_sem, 2)

        xl = x_ref[0].astype(jnp.bfloat16)
        wq = wq_ref[...].astype(jnp.bfloat16)
        wk = wk_ref[...].astype(jnp.bfloat16)
        wv = wv_ref[...].astype(jnp.bfloat16)
        q = jnp.dot(xl, wq, preferred_element_type=jnp.float32)
        k = jnp.dot(xl, wk, preferred_element_type=jnp.float32)
        v = jnp.dot(xl, wv, preferred_element_type=jnp.float32)

        col = lax.broadcasted_iota(jnp.int32, (S_LOC, D), 1)
        lane = lax.rem(col, DH)
        half = lane // 2
        freq = jnp.exp(half.astype(jnp.float32) * (-9.210340371976184 / 64.0))
        row = lax.broadcasted_iota(jnp.int32, (S_LOC, D), 0)
        pos = (row + my * S_LOC).astype(jnp.float32)
        ang = pos * freq
        cosv = jnp.cos(ang)
        sinv = jnp.sin(ang)
        even = lax.rem(col, 2) == 0

        def rope(t):
            t_l = jnp.roll(t, -1, axis=1)
            t_r = jnp.roll(t, 1, axis=1)
            rot = jnp.where(even, -t_l, t_r)
            return t * cosv + rot * sinv

        q_r = rope(q).astype(jnp.bfloat16)
        k_r = rope(k).astype(jnp.bfloat16)
        v_b = v.astype(jnp.bfloat16)

        kall[pl.ds(my * S_LOC, S_LOC), :] = k_r
        vall[pl.ds(my * S_LOC, S_LOC), :] = v_b
        comm[0, pl.ds(0, S_LOC), :] = k_r
        comm[0, pl.ds(S_LOC, S_LOC), :] = v_b

        for h in range(N_DEV - 1):
            send_slot = h % 2
            recv_slot = (h + 1) % 2
            rdma = pltpu.make_async_remote_copy(
                src_ref=comm.at[send_slot],
                dst_ref=comm.at[recv_slot],
                send_sem=send_sems.at[send_slot],
                recv_sem=recv_sems.at[recv_slot],
                device_id=(right,),
                device_id_type=pl.DeviceIdType.MESH,
            )
            rdma.start()
            rdma.wait()
            origin = lax.rem(my - h - 1 + N_DEV, N_DEV)
            kall[pl.ds(origin * S_LOC, S_LOC), :] = comm[recv_slot, pl.ds(0, S_LOC), :]
            vall[pl.ds(origin * S_LOC, S_LOC), :] = comm[recv_slot, pl.ds(S_LOC, S_LOC), :]

        ctx_heads = []
        for hd in range(HQ):
            q_h = q_r[:, hd * DH:(hd + 1) * DH]
            k_h = kall[:, hd * DH:(hd + 1) * DH]
            s = lax.dot_general(
                q_h, k_h, (((1,), (1,)), ((), ())),
                preferred_element_type=jnp.float32,
            ) * SCALE
            m = jnp.max(s, axis=1, keepdims=True)
            p = jnp.exp(s - m)
            l = jnp.sum(p, axis=1, keepdims=True)
            w = (p / l).astype(jnp.bfloat16)
            ctx_h = jnp.dot(w, vall[:, hd * DH:(hd + 1) * DH],
                            preferred_element_type=jnp.float32)
            ctx_heads.append(ctx_h.astype(jnp.bfloat16))
        ctx = jnp.concatenate(ctx_heads, axis=1)

        wo = wo_ref[...].astype(jnp.bfloat16)
        out_ref[0] = jnp.dot(ctx, wo, preferred_element_type=jnp.float32)

    return pl.pallas_call(
        body,
        out_shape=jax.ShapeDtypeStruct((1, S_LOC, D), jnp.float32),
        in_specs=[pl.BlockSpec(memory_space=pltpu.VMEM)] * 5,
        out_specs=pl.BlockSpec(memory_space=pltpu.VMEM),
        scratch_shapes=[
            pltpu.VMEM((S_GLB, D), jnp.bfloat16),
            pltpu.VMEM((S_GLB, D), jnp.bfloat16),
            pltpu.VMEM((2, 2 * S_LOC, D), jnp.bfloat16),
            pltpu.SemaphoreType.DMA((2,)),
            pltpu.SemaphoreType.DMA((2,)),
        ],
        compiler_params=pltpu.CompilerParams(
            collective_id=0,
            vmem_limit_bytes=128 * 1024 * 1024,
        ),
    )(x, Wq, Wk, Wv, Wo)


# baseline (device time: 260550 ns/iter reference)
import jax
import jax.numpy as jnp
from jax import lax
from jax.experimental import pallas as pl
from jax.experimental.pallas import tpu as pltpu

N_DEV = 4
S_LOC = 1024
D = 1024
HQ = 8
DH = 128
SCALE = 0.08838834764831843
NEG_INF = -1e30


def kernel(x, Wq, Wk, Wv, Wo):
    x = x.astype(jnp.bfloat16)
    Wq = Wq.astype(jnp.bfloat16)
    Wk = Wk.astype(jnp.bfloat16)
    Wv = Wv.astype(jnp.bfloat16)
    Wo = Wo.astype(jnp.bfloat16)

    def body(x_ref, wq_ref, wk_ref, wv_ref, wo_ref, out_ref,
             comm, qbuf, ctx, send_sems, recv_sems):
        my = lax.axis_index("i")
        left = lax.rem(my - 1 + N_DEV, N_DEV)
        right = lax.rem(my + 1, N_DEV)

        barrier_sem = pltpu.get_barrier_semaphore()
        for nbr in [left, right]:
            pl.semaphore_signal(
                barrier_sem, inc=1,
                device_id=(nbr,), device_id_type=pl.DeviceIdType.MESH,
            )
        pl.semaphore_wait(barrier_sem, 2)

        lane = lax.broadcasted_iota(jnp.int32, (S_LOC, DH), 1)
        half = lane // 2
        freq = jnp.exp(half.astype(jnp.float32) * (-9.210340371976184 / 64.0))
        row = lax.broadcasted_iota(jnp.int32, (S_LOC, DH), 0)
        pos = (row + my * S_LOC).astype(jnp.float32)
        ang = pos * freq
        cosv = jnp.cos(ang)
        sinv = jnp.sin(ang)
        even = lax.rem(lane, 2) == 0

        def rope(t):
            t_l = jnp.roll(t, -1, axis=1)
            t_r = jnp.roll(t, 1, axis=1)
            rot = jnp.where(even, -t_l, t_r)
            return t * cosv + rot * sinv

        def qkv_head(hd, _):
            cols = pl.ds(hd * DH, DH)
            xl = x_ref[0]
            q_h = jnp.dot(xl, wq_ref[:, cols],
                          preferred_element_type=jnp.float32)
            qbuf[:, cols] = rope(q_h).astype(jnp.bfloat16)
            k_h = jnp.dot(xl, wk_ref[:, cols],
                          preferred_element_type=jnp.float32)
            comm[0, pl.ds(0, S_LOC), cols] = rope(k_h).astype(jnp.bfloat16)
            v_h = jnp.dot(xl, wv_ref[:, cols],
                          preferred_element_type=jnp.float32)
            comm[0, pl.ds(S_LOC, S_LOC), cols] = v_h.astype(jnp.bfloat16)
            return 0

        lax.fori_loop(0, HQ, qkv_head, 0)

        for h in range(N_DEV - 1):
            rdma = pltpu.make_async_remote_copy(
                src_ref=comm.at[h],
                dst_ref=comm.at[h + 1],
                send_sem=send_sems.at[h],
                recv_sem=recv_sems.at[h + 1],
                device_id=(right,),
                device_id_type=pl.DeviceIdType.MESH,
            )
            rdma.start()
            rdma.wait()

        def attn_head(hd, _):
            cols = pl.ds(hd * DH, DH)
            q_h = qbuf[:, cols]

            def chunk(c, carry):
                m_run, l_run, acc = carry
                k_c = comm[c, pl.ds(0, S_LOC), cols]
                v_c = comm[c, pl.ds(S_LOC, S_LOC), cols]
                s = lax.dot_general(
                    q_h, k_c, (((1,), (1,)), ((), ())),
                    preferred_element_type=jnp.float32,
                ) * SCALE
                m_new = jnp.maximum(m_run, jnp.max(s, axis=1, keepdims=True))
                alpha = jnp.exp(m_run - m_new)
                p = jnp.exp(s - m_new)
                l_new = l_run * alpha + jnp.sum(p, axis=1, keepdims=True)
                acc = acc * alpha + jnp.dot(
                    p.astype(jnp.bfloat16), v_c,
                    preferred_element_type=jnp.float32)
                return m_new, l_new, acc

            m0 = jnp.full((S_LOC, 1), NEG_INF, jnp.float32)
            l0 = jnp.zeros((S_LOC, 1), jnp.float32)
            a0 = jnp.zeros((S_LOC, DH), jnp.float32)
            _, l_fin, acc_fin = lax.fori_loop(0, N_DEV, chunk, (m0, l0, a0))
            ctx[:, cols] = (acc_fin / l_fin).astype(jnp.bfloat16)
            return 0

        lax.fori_loop(0, HQ, attn_head, 0)

        out_ref[0] = jnp.dot(ctx[...], wo_ref[...],
                             preferred_element_type=jnp.float32)

    return pl.pallas_call(
        body,
        out_shape=jax.ShapeDtypeStruct((1, S_LOC, D), jnp.float32),
        in_specs=[pl.BlockSpec(memory_space=pltpu.VMEM)] * 5,
        out_specs=pl.BlockSpec(memory_space=pltpu.VMEM),
        scratch_shapes=[
            pltpu.VMEM((N_DEV, 2 * S_LOC, D), jnp.bfloat16),
            pltpu.VMEM((S_LOC, D), jnp.bfloat16),
            pltpu.VMEM((S_LOC, D), jnp.bfloat16),
            pltpu.SemaphoreType.DMA((N_DEV - 1,)),
            pltpu.SemaphoreType.DMA((N_DEV,)),
        ],
        compiler_params=pltpu.CompilerParams(
            collective_id=0,
            vmem_limit_bytes=128 * 1024 * 1024,
        ),
    )(x, Wq, Wk, Wv, Wo)


# device time: 213020 ns/iter; 1.2231x vs baseline; 1.2231x over previous
import jax
import jax.numpy as jnp
from jax import lax
from jax.experimental import pallas as pl
from jax.experimental.pallas import tpu as pltpu

N_DEV = 4
S_LOC = 1024
D = 1024
HQ = 8
DH = 128
SCALE = 0.08838834764831843
NEG_INF = -1e30


def kernel(x, Wq, Wk, Wv, Wo):
    x = x.astype(jnp.bfloat16)
    Wq = Wq.astype(jnp.bfloat16)
    Wk = Wk.astype(jnp.bfloat16)
    Wv = Wv.astype(jnp.bfloat16)
    Wo = Wo.astype(jnp.bfloat16)

    def body(x_ref, wq_ref, wk_ref, wv_ref, wo_ref, out_ref,
             comm, qbuf, ctx, macc, lacc, accf, send_sems, recv_sems):
        my = lax.axis_index("i")
        left = lax.rem(my - 1 + N_DEV, N_DEV)
        right = lax.rem(my + 1, N_DEV)

        barrier_sem = pltpu.get_barrier_semaphore()
        for nbr in [left, right]:
            pl.semaphore_signal(
                barrier_sem, inc=1,
                device_id=(nbr,), device_id_type=pl.DeviceIdType.MESH,
            )
        pl.semaphore_wait(barrier_sem, 2)

        lane = lax.broadcasted_iota(jnp.int32, (S_LOC, DH), 1)
        half = lane // 2
        freq = jnp.exp(half.astype(jnp.float32) * (-9.210340371976184 / 64.0))
        row = lax.broadcasted_iota(jnp.int32, (S_LOC, DH), 0)
        pos = (row + my * S_LOC).astype(jnp.float32)
        ang = pos * freq
        cosv = jnp.cos(ang)
        sinv = jnp.sin(ang)
        even = lax.rem(lane, 2) == 0

        def rope(t):
            t_l = jnp.roll(t, -1, axis=1)
            t_r = jnp.roll(t, 1, axis=1)
            rot = jnp.where(even, -t_l, t_r)
            return t * cosv + rot * sinv

        def qkv_head(hd, _):
            cols = pl.ds(hd * DH, DH)
            xl = x_ref[0]
            q_h = jnp.dot(xl, wq_ref[:, cols],
                          preferred_element_type=jnp.float32)
            qbuf[:, cols] = rope(q_h).astype(jnp.bfloat16)
            k_h = jnp.dot(xl, wk_ref[:, cols],
                          preferred_element_type=jnp.float32)
            comm[0, pl.ds(0, S_LOC), cols] = rope(k_h).astype(jnp.bfloat16)
            v_h = jnp.dot(xl, wv_ref[:, cols],
                          preferred_element_type=jnp.float32)
            comm[0, pl.ds(S_LOC, S_LOC), cols] = v_h.astype(jnp.bfloat16)
            return 0

        lax.fori_loop(0, HQ, qkv_head, 0)

        macc[...] = jnp.full((HQ, S_LOC, 1), NEG_INF, jnp.float32)
        lacc[...] = jnp.zeros((HQ, S_LOC, 1), jnp.float32)
        accf[...] = jnp.zeros((S_LOC, D), jnp.float32)

        def attn_chunk(c):
            def head(hd, _):
                cols = pl.ds(hd * DH, DH)
                q_h = qbuf[:, cols]
                k_c = comm[c, pl.ds(0, S_LOC), cols]
                v_c = comm[c, pl.ds(S_LOC, S_LOC), cols]
                s = lax.dot_general(
                    q_h, k_c, (((1,), (1,)), ((), ())),
                    preferred_element_type=jnp.float32,
                ) * SCALE
                m_old = macc[hd]
                m_new = jnp.maximum(m_old, jnp.max(s, axis=1, keepdims=True))
                alpha = jnp.exp(m_old - m_new)
                p = jnp.exp(s - m_new)
                lacc[hd] = lacc[hd] * alpha + jnp.sum(p, axis=1, keepdims=True)
                accf[:, cols] = accf[:, cols] * alpha + jnp.dot(
                    p.astype(jnp.bfloat16), v_c,
                    preferred_element_type=jnp.float32)
                macc[hd] = m_new
                return 0

            lax.fori_loop(0, HQ, head, 0)

        def hop(h):
            return pltpu.make_async_remote_copy(
                src_ref=comm.at[h],
                dst_ref=comm.at[h + 1],
                send_sem=send_sems.at[h],
                recv_sem=recv_sems.at[h + 1],
                device_id=(right,),
                device_id_type=pl.DeviceIdType.MESH,
            )

        hop(0).start()
        for c in range(N_DEV):
            attn_chunk(c)
            if c < N_DEV - 1:
                hop(c).wait()
                if c < N_DEV - 2:
                    hop(c + 1).start()

        def norm_head(hd, _):
            cols = pl.ds(hd * DH, DH)
            ctx[:, cols] = (accf[:, cols] / lacc[hd]).astype(jnp.bfloat16)
            return 0

        lax.fori_loop(0, HQ, norm_head, 0)

        out_ref[0] = jnp.dot(ctx[...], wo_ref[...],
                             preferred_element_type=jnp.float32)

    return pl.pallas_call(
        body,
        out_shape=jax.ShapeDtypeStruct((1, S_LOC, D), jnp.float32),
        in_specs=[pl.BlockSpec(memory_space=pltpu.VMEM)] * 5,
        out_specs=pl.BlockSpec(memory_space=pltpu.VMEM),
        scratch_shapes=[
            pltpu.VMEM((N_DEV, 2 * S_LOC, D), jnp.bfloat16),
            pltpu.VMEM((S_LOC, D), jnp.bfloat16),
            pltpu.VMEM((S_LOC, D), jnp.bfloat16),
            pltpu.VMEM((HQ, S_LOC, 1), jnp.float32),
            pltpu.VMEM((HQ, S_LOC, 1), jnp.float32),
            pltpu.VMEM((S_LOC, D), jnp.float32),
            pltpu.SemaphoreType.DMA((N_DEV - 1,)),
            pltpu.SemaphoreType.DMA((N_DEV,)),
        ],
        compiler_params=pltpu.CompilerParams(
            collective_id=0,
            vmem_limit_bytes=128 * 1024 * 1024,
        ),
    )(x, Wq, Wk, Wv, Wo)


# device time: 163129 ns/iter; 1.5972x vs baseline; 1.3058x over previous
import jax
import jax.numpy as jnp
from jax import lax
from jax.experimental import pallas as pl
from jax.experimental.pallas import tpu as pltpu

N_DEV = 4
S_LOC = 1024
D = 1024
HQ = 8
DH = 128
SCALE = 0.08838834764831843
NEG_INF = -1e30


def kernel(x, Wq, Wk, Wv, Wo):
    x = x.astype(jnp.bfloat16)
    Wq = Wq.astype(jnp.bfloat16)
    Wk = Wk.astype(jnp.bfloat16)
    Wv = Wv.astype(jnp.bfloat16)
    Wo = Wo.astype(jnp.bfloat16)

    def body(x_ref, wq_ref, wk_ref, wv_ref, wo_ref, out_ref,
             comm, qbuf, ctx, macc, lacc, accf, send_sems, recv_sems):
        my = lax.axis_index("i")
        left = lax.rem(my - 1 + N_DEV, N_DEV)
        right = lax.rem(my + 1, N_DEV)

        barrier_sem = pltpu.get_barrier_semaphore()
        for nbr in [left, right]:
            pl.semaphore_signal(
                barrier_sem, inc=1,
                device_id=(nbr,), device_id_type=pl.DeviceIdType.MESH,
            )
        pl.semaphore_wait(barrier_sem, 2)

        lane = lax.broadcasted_iota(jnp.int32, (S_LOC, DH), 1)
        half = lane // 2
        freq = jnp.exp(half.astype(jnp.float32) * (-9.210340371976184 / 64.0))
        row = lax.broadcasted_iota(jnp.int32, (S_LOC, DH), 0)
        pos = (row + my * S_LOC).astype(jnp.float32)
        ang = pos * freq
        cosv = jnp.cos(ang)
        sinv = jnp.sin(ang)
        even = lax.rem(lane, 2) == 0

        def rope(t):
            t_l = jnp.roll(t, -1, axis=1)
            t_r = jnp.roll(t, 1, axis=1)
            rot = jnp.where(even, -t_l, t_r)
            return t * cosv + rot * sinv

        def qkv_head(hd, _):
            cols = pl.ds(hd * DH, DH)
            xl = x_ref[0]
            q_h = jnp.dot(xl, wq_ref[:, cols],
                          preferred_element_type=jnp.float32)
            qbuf[:, cols] = rope(q_h).astype(jnp.bfloat16)
            k_h = jnp.dot(xl, wk_ref[:, cols],
                          preferred_element_type=jnp.float32)
            comm[0, pl.ds(0, S_LOC), cols] = rope(k_h).astype(jnp.bfloat16)
            v_h = jnp.dot(xl, wv_ref[:, cols],
                          preferred_element_type=jnp.float32)
            comm[0, pl.ds(S_LOC, S_LOC), cols] = v_h.astype(jnp.bfloat16)
            return 0

        lax.fori_loop(0, HQ, qkv_head, 0)

        macc[...] = jnp.full((HQ, S_LOC, 1), NEG_INF, jnp.float32)
        lacc[...] = jnp.zeros((HQ, S_LOC, 1), jnp.float32)
        accf[...] = jnp.zeros((S_LOC, D), jnp.float32)

        def attn_chunk(c):
            def head(hd, _):
                cols = pl.ds(hd * DH, DH)
                q_h = qbuf[:, cols]
                k_c = comm[c, pl.ds(0, S_LOC), cols]
                v_c = comm[c, pl.ds(S_LOC, S_LOC), cols]
                s = lax.dot_general(
                    q_h, k_c, (((1,), (1,)), ((), ())),
                    preferred_element_type=jnp.float32,
                ) * SCALE
                m_old = macc[hd]
                m_new = jnp.maximum(m_old, jnp.max(s, axis=1, keepdims=True))
                alpha = jnp.exp(m_old - m_new)
                p = jnp.exp(s - m_new)
                lacc[hd] = lacc[hd] * alpha + jnp.sum(p, axis=1, keepdims=True)
                accf[:, cols] = accf[:, cols] * alpha + jnp.dot(
                    p.astype(jnp.bfloat16), v_c,
                    preferred_element_type=jnp.float32)
                macc[hd] = m_new
                return 0

            lax.fori_loop(0, HQ, head, 0)

        krows = pl.ds(0, S_LOC)
        vrows = pl.ds(S_LOC, S_LOC)
        r1 = pltpu.make_async_remote_copy(
            src_ref=comm.at[0], dst_ref=comm.at[1],
            send_sem=send_sems.at[0], recv_sem=recv_sems.at[1],
            device_id=(right,), device_id_type=pl.DeviceIdType.MESH,
        )
        l1 = pltpu.make_async_remote_copy(
            src_ref=comm.at[0], dst_ref=comm.at[2],
            send_sem=send_sems.at[1], recv_sem=recv_sems.at[2],
            device_id=(left,), device_id_type=pl.DeviceIdType.MESH,
        )
        r2 = pltpu.make_async_remote_copy(
            src_ref=comm.at[1, krows, :], dst_ref=comm.at[3, krows, :],
            send_sem=send_sems.at[2], recv_sem=recv_sems.at[3],
            device_id=(right,), device_id_type=pl.DeviceIdType.MESH,
        )
        l2 = pltpu.make_async_remote_copy(
            src_ref=comm.at[2, vrows, :], dst_ref=comm.at[3, vrows, :],
            send_sem=send_sems.at[3], recv_sem=recv_sems.at[4],
            device_id=(left,), device_id_type=pl.DeviceIdType.MESH,
        )

        r1.start()
        l1.start()
        attn_chunk(0)
        r1.wait()
        r2.start()
        l1.wait()
        l2.start()
        attn_chunk(1)
        attn_chunk(2)
        r2.wait()
        l2.wait()
        attn_chunk(3)

        def norm_head(hd, _):
            cols = pl.ds(hd * DH, DH)
            ctx[:, cols] = (accf[:, cols] / lacc[hd]).astype(jnp.bfloat16)
            return 0

        lax.fori_loop(0, HQ, norm_head, 0)

        out_ref[0] = jnp.dot(ctx[...], wo_ref[...],
                             preferred_element_type=jnp.float32)

    return pl.pallas_call(
        body,
        out_shape=jax.ShapeDtypeStruct((1, S_LOC, D), jnp.float32),
        in_specs=[pl.BlockSpec(memory_space=pltpu.VMEM)] * 5,
        out_specs=pl.BlockSpec(memory_space=pltpu.VMEM),
        scratch_shapes=[
            pltpu.VMEM((N_DEV, 2 * S_LOC, D), jnp.bfloat16),
            pltpu.VMEM((S_LOC, D), jnp.bfloat16),
            pltpu.VMEM((S_LOC, D), jnp.bfloat16),
            pltpu.VMEM((HQ, S_LOC, 1), jnp.float32),
            pltpu.VMEM((HQ, S_LOC, 1), jnp.float32),
            pltpu.VMEM((S_LOC, D), jnp.float32),
            pltpu.SemaphoreType.DMA((4,)),
            pltpu.SemaphoreType.DMA((5,)),
        ],
        compiler_params=pltpu.CompilerParams(
            collective_id=0,
            vmem_limit_bytes=128 * 1024 * 1024,
        ),
    )(x, Wq, Wk, Wv, Wo)


# device time: 132839 ns/iter; 1.9614x vs baseline; 1.2280x over previous
import jax
import jax.numpy as jnp
from jax import lax
from jax.experimental import pallas as pl
from jax.experimental.pallas import tpu as pltpu

N_DEV = 4
S_LOC = 1024
D = 1024
HQ = 8
DH = 128
SCALE = 0.08838834764831843
NEG_INF = -1e30


def kernel(x, Wq, Wk, Wv, Wo):
    x = x.astype(jnp.bfloat16)
    Wq = Wq.astype(jnp.bfloat16)
    Wk = Wk.astype(jnp.bfloat16)
    Wv = Wv.astype(jnp.bfloat16)
    Wo = Wo.astype(jnp.bfloat16)

    def body(x_ref, wq_ref, wk_ref, wv_ref, wo_ref, out_ref,
             comm, qbuf, ctx, lacc, accf, send_sems, recv_sems):
        my = lax.axis_index("i")
        left = lax.rem(my - 1 + N_DEV, N_DEV)
        right = lax.rem(my + 1, N_DEV)

        barrier_sem = pltpu.get_barrier_semaphore()
        for nbr in [left, right]:
            pl.semaphore_signal(
                barrier_sem, inc=1,
                device_id=(nbr,), device_id_type=pl.DeviceIdType.MESH,
            )
        pl.semaphore_wait(barrier_sem, 2)

        lane = lax.broadcasted_iota(jnp.int32, (S_LOC, DH), 1)
        half = lane // 2
        freq = jnp.exp(half.astype(jnp.float32) * (-9.210340371976184 / 64.0))
        row = lax.broadcasted_iota(jnp.int32, (S_LOC, DH), 0)
        pos = (row + my * S_LOC).astype(jnp.float32)
        ang = pos * freq
        cosv = jnp.cos(ang)
        sinv = jnp.sin(ang)
        even = lax.rem(lane, 2) == 0

        def rope(t):
            t_l = jnp.roll(t, -1, axis=1)
            t_r = jnp.roll(t, 1, axis=1)
            rot = jnp.where(even, -t_l, t_r)
            return t * cosv + rot * sinv

        def qkv_head(hd, _):
            cols = pl.ds(hd * DH, DH)
            xl = x_ref[0]
            q_h = jnp.dot(xl, wq_ref[:, cols],
                          preferred_element_type=jnp.float32)
            qbuf[:, cols] = (rope(q_h) * SCALE).astype(jnp.bfloat16)
            k_h = jnp.dot(xl, wk_ref[:, cols],
                          preferred_element_type=jnp.float32)
            comm[0, pl.ds(0, S_LOC), cols] = rope(k_h).astype(jnp.bfloat16)
            v_h = jnp.dot(xl, wv_ref[:, cols],
                          preferred_element_type=jnp.float32)
            comm[0, pl.ds(S_LOC, S_LOC), cols] = v_h.astype(jnp.bfloat16)
            return 0

        lax.fori_loop(0, HQ, qkv_head, 0)

        lacc[...] = jnp.zeros((HQ, S_LOC, 1), jnp.float32)
        accf[...] = jnp.zeros((S_LOC, D), jnp.float32)

        def attn_chunk(c):
            def head(hd, _):
                cols = pl.ds(hd * DH, DH)
                q_h = qbuf[:, cols]
                k_c = comm[c, pl.ds(0, S_LOC), cols]
                v_c = comm[c, pl.ds(S_LOC, S_LOC), cols]
                s = lax.dot_general(
                    q_h, k_c, (((1,), (1,)), ((), ())),
                    preferred_element_type=jnp.float32,
                )
                p = jnp.exp(s)
                lacc[hd] = lacc[hd] + jnp.sum(p, axis=1, keepdims=True)
                accf[:, cols] = accf[:, cols] + jnp.dot(
                    p.astype(jnp.bfloat16), v_c,
                    preferred_element_type=jnp.float32)
                return 0

            lax.fori_loop(0, HQ, head, 0)

        krows = pl.ds(0, S_LOC)
        vrows = pl.ds(S_LOC, S_LOC)
        r1 = pltpu.make_async_remote_copy(
            src_ref=comm.at[0], dst_ref=comm.at[1],
            send_sem=send_sems.at[0], recv_sem=recv_sems.at[1],
            device_id=(right,), device_id_type=pl.DeviceIdType.MESH,
        )
        l1 = pltpu.make_async_remote_copy(
            src_ref=comm.at[0], dst_ref=comm.at[2],
            send_sem=send_sems.at[1], recv_sem=recv_sems.at[2],
            device_id=(left,), device_id_type=pl.DeviceIdType.MESH,
        )
        r2 = pltpu.make_async_remote_copy(
            src_ref=comm.at[1, krows, :], dst_ref=comm.at[3, krows, :],
            send_sem=send_sems.at[2], recv_sem=recv_sems.at[3],
            device_id=(right,), device_id_type=pl.DeviceIdType.MESH,
        )
        l2 = pltpu.make_async_remote_copy(
            src_ref=comm.at[2, vrows, :], dst_ref=comm.at[3, vrows, :],
            send_sem=send_sems.at[3], recv_sem=recv_sems.at[4],
            device_id=(left,), device_id_type=pl.DeviceIdType.MESH,
        )

        r1.start()
        l1.start()
        attn_chunk(0)
        r1.wait()
        r2.start()
        l1.wait()
        l2.start()
        attn_chunk(1)
        attn_chunk(2)
        r2.wait()
        l2.wait()
        attn_chunk(3)

        def norm_head(hd, _):
            cols = pl.ds(hd * DH, DH)
            ctx[:, cols] = (accf[:, cols] / lacc[hd]).astype(jnp.bfloat16)
            return 0

        lax.fori_loop(0, HQ, norm_head, 0)

        out_ref[0] = jnp.dot(ctx[...], wo_ref[...],
                             preferred_element_type=jnp.float32)

    return pl.pallas_call(
        body,
        out_shape=jax.ShapeDtypeStruct((1, S_LOC, D), jnp.float32),
        in_specs=[pl.BlockSpec(memory_space=pltpu.VMEM)] * 5,
        out_specs=pl.BlockSpec(memory_space=pltpu.VMEM),
        scratch_shapes=[
            pltpu.VMEM((N_DEV, 2 * S_LOC, D), jnp.bfloat16),
            pltpu.VMEM((S_LOC, D), jnp.bfloat16),
            pltpu.VMEM((S_LOC, D), jnp.bfloat16),
            pltpu.VMEM((HQ, S_LOC, 1), jnp.float32),
            pltpu.VMEM((S_LOC, D), jnp.float32),
            pltpu.SemaphoreType.DMA((4,)),
            pltpu.SemaphoreType.DMA((5,)),
        ],
        compiler_params=pltpu.CompilerParams(
            collective_id=0,
            vmem_limit_bytes=128 * 1024 * 1024,
        ),
    )(x, Wq, Wk, Wv, Wo)


# device time: 123782 ns/iter; 2.1049x vs baseline; 1.0732x over previous
import jax
import jax.numpy as jnp
from jax import lax
from jax.experimental import pallas as pl
from jax.experimental.pallas import tpu as pltpu

N_DEV = 4
S_LOC = 1024
D = 1024
HQ = 8
DH = 128
SCALE = 0.08838834764831843
NEG_INF = -1e30


def kernel(x, Wq, Wk, Wv, Wo):
    x = x.astype(jnp.bfloat16)
    Wq = Wq.astype(jnp.bfloat16)
    Wk = Wk.astype(jnp.bfloat16)
    Wv = Wv.astype(jnp.bfloat16)
    Wo = Wo.astype(jnp.bfloat16)

    def body(x_ref, wq_ref, wk_ref, wv_ref, wo_ref, out_ref,
             comm, qbuf, ctx, lacc, accf, send_sems, recv_sems):
        my = lax.axis_index("i")
        left = lax.rem(my - 1 + N_DEV, N_DEV)
        right = lax.rem(my + 1, N_DEV)

        barrier_sem = pltpu.get_barrier_semaphore()
        for nbr in [left, right]:
            pl.semaphore_signal(
                barrier_sem, inc=1,
                device_id=(nbr,), device_id_type=pl.DeviceIdType.MESH,
            )
        pl.semaphore_wait(barrier_sem, 2)

        lane = lax.broadcasted_iota(jnp.int32, (S_LOC, DH), 1)
        half = lane // 2
        freq = jnp.exp(half.astype(jnp.float32) * (-9.210340371976184 / 64.0))
        row = lax.broadcasted_iota(jnp.int32, (S_LOC, DH), 0)
        pos = (row + my * S_LOC).astype(jnp.float32)
        ang = pos * freq
        cosv = jnp.cos(ang)
        sinv = jnp.sin(ang)
        even = lax.rem(lane, 2) == 0

        def rope(t):
            t_l = jnp.roll(t, -1, axis=1)
            t_r = jnp.roll(t, 1, axis=1)
            rot = jnp.where(even, -t_l, t_r)
            return t * cosv + rot * sinv

        def k_head(hd, _):
            cols = pl.ds(hd * DH, DH)
            k_h = jnp.dot(x_ref[0], wk_ref[:, cols],
                          preferred_element_type=jnp.float32)
            comm[0, pl.ds(0, S_LOC), cols] = rope(k_h).astype(jnp.bfloat16)
            return 0

        def v_head(hd, _):
            cols = pl.ds(hd * DH, DH)
            v_h = jnp.dot(x_ref[0], wv_ref[:, cols],
                          preferred_element_type=jnp.float32)
            comm[0, pl.ds(S_LOC, S_LOC), cols] = v_h.astype(jnp.bfloat16)
            return 0

        def q_head(hd, _):
            cols = pl.ds(hd * DH, DH)
            q_h = jnp.dot(x_ref[0], wq_ref[:, cols],
                          preferred_element_type=jnp.float32)
            qbuf[:, cols] = (rope(q_h) * SCALE).astype(jnp.bfloat16)
            return 0

        lacc[...] = jnp.zeros((HQ, S_LOC, 1), jnp.float32)
        accf[...] = jnp.zeros((S_LOC, D), jnp.float32)

        def attn_chunk(c):
            def head(hd, _):
                cols = pl.ds(hd * DH, DH)
                q_h = qbuf[:, cols]
                k_c = comm[c, pl.ds(0, S_LOC), cols]
                v_c = comm[c, pl.ds(S_LOC, S_LOC), cols]
                s = lax.dot_general(
                    q_h, k_c, (((1,), (1,)), ((), ())),
                    preferred_element_type=jnp.float32,
                )
                p = jnp.exp(s)
                lacc[hd] = lacc[hd] + jnp.sum(p, axis=1, keepdims=True)
                accf[:, cols] = accf[:, cols] + jnp.dot(
                    p.astype(jnp.bfloat16), v_c,
                    preferred_element_type=jnp.float32)
                return 0

            lax.fori_loop(0, HQ, head, 0)

        krows = pl.ds(0, S_LOC)
        vrows = pl.ds(S_LOC, S_LOC)

        def xfer(slot_src, slot_dst, rows, sem, target):
            return pltpu.make_async_remote_copy(
                src_ref=comm.at[slot_src, rows, :],
                dst_ref=comm.at[slot_dst, rows, :],
                send_sem=send_sems.at[sem], recv_sem=recv_sems.at[sem],
                device_id=(target,), device_id_type=pl.DeviceIdType.MESH,
            )

        r1k = xfer(0, 1, krows, 0, right)
        l1k = xfer(0, 2, krows, 1, left)
        r1v = xfer(0, 1, vrows, 2, right)
        l1v = xfer(0, 2, vrows, 3, left)
        r2 = xfer(1, 3, krows, 4, right)
        l2 = xfer(2, 3, vrows, 5, left)

        lax.fori_loop(0, HQ, k_head, 0)
        r1k.start()
        l1k.start()
        lax.fori_loop(0, HQ, v_head, 0)
        r1v.start()
        l1v.start()
        lax.fori_loop(0, HQ, q_head, 0)
        attn_chunk(0)
        r1k.wait()
        r2.start()
        l1v.wait()
        l2.start()
        r1v.wait()
        attn_chunk(1)
        l1k.wait()
        attn_chunk(2)
        r2.wait()
        l2.wait()
        attn_chunk(3)

        def norm_head(hd, _):
            cols = pl.ds(hd * DH, DH)
            ctx[:, cols] = (accf[:, cols] / lacc[hd]).astype(jnp.bfloat16)
            return 0

        lax.fori_loop(0, HQ, norm_head, 0)

        out_ref[0] = jnp.dot(ctx[...], wo_ref[...],
                             preferred_element_type=jnp.float32)

    return pl.pallas_call(
        body,
        out_shape=jax.ShapeDtypeStruct((1, S_LOC, D), jnp.float32),
        in_specs=[pl.BlockSpec(memory_space=pltpu.VMEM)] * 5,
        out_specs=pl.BlockSpec(memory_space=pltpu.VMEM),
        scratch_shapes=[
            pltpu.VMEM((N_DEV, 2 * S_LOC, D), jnp.bfloat16),
            pltpu.VMEM((S_LOC, D), jnp.bfloat16),
            pltpu.VMEM((S_LOC, D), jnp.bfloat16),
            pltpu.VMEM((HQ, S_LOC, 1), jnp.float32),
            pltpu.VMEM((S_LOC, D), jnp.float32),
            pltpu.SemaphoreType.DMA((6,)),
            pltpu.SemaphoreType.DMA((6,)),
        ],
        compiler_params=pltpu.CompilerParams(
            collective_id=0,
            vmem_limit_bytes=128 * 1024 * 1024,
        ),
    )(x, Wq, Wk, Wv, Wo)
